# baseline (speedup 1.0000x reference)
import dataclasses
import functools
import jax
import jax.numpy as jnp
from jax import lax
from jax.experimental import pallas as pl
from jax.experimental.pallas import tpu as pltpu
from jax.experimental.pallas import tpu_sc as plsc

N = 10000
E = 320000
D = 128
NC = 2
NS = 16
NW = NC * NS
EPW = E // NW
CHUNK = 80
NCHUNK = EPW // CHUNK
RPT = 624
RREM = N - (NS - 1) * RPT - RPT


_GDN = lax.GatherDimensionNumbers(
    offset_dims=(), collapsed_slice_dims=(0,), start_index_map=(0,))


def _bcast_lane(v, r):
    idx = jnp.full((16, 1), r, jnp.int32)
    return lax.gather(v, idx, _GDN, slice_sizes=(1,),
                      mode=lax.GatherScatterMode.PROMISE_IN_BOUNDS)


def _stats_body(asrc_hbm, adst_hbm, src_hbm, dst_hbm, exf_hbm, denp_hbm,
                asrc_v, adst_v, den_v, sidx_v, didx_v, ex_v):
    cid = lax.axis_index("c")
    sid = lax.axis_index("s")
    wid = cid * NS + sid

    pltpu.sync_copy(asrc_hbm, asrc_v)
    pltpu.sync_copy(adst_hbm, adst_v)
    pltpu.sync_copy(src_hbm.at[pl.ds(wid * EPW, EPW)], sidx_v)
    pltpu.sync_copy(dst_hbm.at[pl.ds(wid * EPW, EPW)], didx_v)

    zeros16 = jnp.zeros((16,), jnp.float32)

    @pl.loop(0, N // 16)
    def _(i):
        den_v[pl.ds(i * 16, 16)] = zeros16

    @pl.loop(0, EPW // 16, unroll=4)
    def _(i):
        si = sidx_v[pl.ds(i * 16, 16)]
        di = didx_v[pl.ds(i * 16, 16)]
        e = plsc.load_gather(asrc_v, [si]) + plsc.load_gather(adst_v, [di])
        e = jnp.maximum(e, 0.2 * e)
        ex = jnp.exp(e)
        plsc.addupdate_scatter(den_v, [di], ex)
        ex_v[pl.ds(i * 16, 16)] = ex

    pltpu.sync_copy(ex_v, exf_hbm.at[pl.ds(wid * EPW, EPW)])
    pltpu.sync_copy(den_v, denp_hbm.at[wid])


def _bcast_lane_dyn(v, r):
    idx = jnp.full((16, 1), r, jnp.int32)
    return lax.gather(v, idx, _GDN, slice_sizes=(1,),
                      mode=lax.GatherScatterMode.PROMISE_IN_BOUNDS)


def _scale_group(exg, grow, srow, g):
    @plsc.parallel_loop(0, 16, unroll=4)
    def _(r):
        k = g * 16 + r
        bc = _bcast_lane_dyn(exg, r)
        for j in range(D // 16):
            srow[k, pl.ds(j * 16, 16)] = grow[k, pl.ds(j * 16, 16)] * bc


def _agg_body(h_hbm, src_hbm, dst_hbm, exf_hbm, zero_hbm, outp_hbm,
              sidx_s0, sidx_s1, didx_s0, didx_s1, ex_s0, ex_s1,
              grow0, grow1, srow0, srow1, shacc,
              gsem0, gsem1, ssem0, ssem1, isem0, isem1):
    cid = lax.axis_index("c")
    sid = lax.axis_index("s")
    wid = cid * NS + sid
    sidx_s = (sidx_s0, sidx_s1)
    didx_s = (didx_s0, didx_s1)
    ex_s = (ex_s0, ex_s1)
    grow = (grow0, grow1)
    srow = (srow0, srow1)
    gsem = (gsem0, gsem1)
    ssem = (ssem0, ssem1)
    isem = (isem0, isem1)
    NG = CHUNK // 16
    base0 = wid * EPW

    pltpu.sync_copy(zero_hbm.at[pl.ds(sid * RPT, RPT)],
                    shacc.at[pl.ds(sid * RPT, RPT)])

    @pl.when(sid == NS - 1)
    def _():
        pltpu.sync_copy(zero_hbm.at[pl.ds(NS * RPT, RREM)],
                        shacc.at[pl.ds(NS * RPT, RREM)])

    zeros16 = jnp.zeros((16,), jnp.float32)
    zeros16i = jnp.zeros((16,), jnp.int32)

    @pl.loop(0, CHUNK)
    def _(k):
        for j in range(D // 16):
            srow0[k, pl.ds(j * 16, 16)] = zeros16
            srow1[k, pl.ds(j * 16, 16)] = zeros16

    for b in range(2):
        pltpu.sync_copy(src_hbm.at[pl.ds(base0 + b * CHUNK, CHUNK)], sidx_s[b])
        pltpu.sync_copy(dst_hbm.at[pl.ds(base0 + b * CHUNK, CHUNK)], didx_s[b])
        pltpu.sync_copy(exf_hbm.at[pl.ds(base0 + b * CHUNK, CHUNK)], ex_s[b])

    plsc.subcore_barrier()

    for b in range(2):
        pltpu.async_copy(h_hbm.at[sidx_s[b]], grow[b], gsem[b])
        for g in range(NG):
            pltpu.async_copy(srow[b].at[pl.ds(g * 16, 16)],
                             shacc.at[zeros16i], ssem[b], add=True)

    def wait_scatter(b):
        for g in range(NG):
            pltpu.make_async_copy(srow[b].at[pl.ds(g * 16, 16)],
                                  shacc.at[zeros16i], ssem[b]).wait()

    def do_chunk(b, t):
        c = 2 * t + b
        ce = base0 + c * CHUNK
        nce = ce + 2 * CHUNK
        more = c + 2 < NCHUNK
        pltpu.make_async_copy(h_hbm.at[sidx_s[b]], grow[b], gsem[b]).wait()

        @pl.when(more)
        def _():
            pltpu.async_copy(src_hbm.at[pl.ds(nce, CHUNK)], sidx_s[b], isem[b])

        wait_scatter(b)
        dig = [didx_s[b][pl.ds(g * 16, 16)] for g in range(NG)]

        @pl.when(more)
        def _():
            pltpu.async_copy(dst_hbm.at[pl.ds(nce, CHUNK)], didx_s[b], isem[b])

        exb = ex_s[b]
        growb = grow[b]
        srowb = srow[b]

        @plsc.parallel_loop(0, CHUNK, unroll=8)
        def _(r):
            bc = plsc.load_gather(exb, [jnp.full((16,), r, jnp.int32)])
            for j in range(D // 16):
                srowb[r, pl.ds(j * 16, 16)] = growb[r, pl.ds(j * 16, 16)] * bc

        for g in range(NG):
            pltpu.async_copy(srow[b].at[pl.ds(g * 16, 16)],
                             shacc.at[dig[g]], ssem[b], add=True)

        @pl.when(more)
        def _():
            pltpu.async_copy(exf_hbm.at[pl.ds(nce, CHUNK)], ex_s[b], isem[b])

        @pl.when(more)
        def _():
            pltpu.make_async_copy(src_hbm.at[pl.ds(nce, CHUNK)], sidx_s[b],
                                  isem[b]).wait()
            pltpu.make_async_copy(dst_hbm.at[pl.ds(nce, CHUNK)], didx_s[b],
                                  isem[b]).wait()
            pltpu.make_async_copy(exf_hbm.at[pl.ds(nce, CHUNK)], ex_s[b],
                                  isem[b]).wait()
            pltpu.async_copy(h_hbm.at[sidx_s[b]], grow[b], gsem[b])

    @pl.loop(0, NCHUNK // 2)
    def _(t):
        do_chunk(0, t)
        do_chunk(1, t)

    do_chunk(0, NCHUNK // 2)

    wait_scatter(1)
    wait_scatter(0)

    plsc.subcore_barrier()
    pltpu.sync_copy(shacc.at[pl.ds(sid * RPT, RPT)],
                    outp_hbm.at[cid].at[pl.ds(sid * RPT, RPT)])

    @pl.when(sid == NS - 1)
    def _():
        pltpu.sync_copy(shacc.at[pl.ds(NS * RPT, RREM)],
                        outp_hbm.at[cid].at[pl.ds(NS * RPT, RREM)])


_sc_params = pltpu.CompilerParams()
if "needs_layout_passes" in pltpu.CompilerParams.__dataclass_fields__:
    _sc_params = dataclasses.replace(_sc_params, needs_layout_passes=False)

_sc_mesh = plsc.VectorSubcoreMesh(core_axis_name="c", subcore_axis_name="s")


@functools.partial(
    pl.kernel,
    out_type=(jax.ShapeDtypeStruct((E,), jnp.float32),
              jax.ShapeDtypeStruct((NW, N), jnp.float32)),
    mesh=_sc_mesh,
    compiler_params=_sc_params,
    scratch_types=[
        pltpu.VMEM((N,), jnp.float32),
        pltpu.VMEM((N,), jnp.float32),
        pltpu.VMEM((N,), jnp.float32),
        pltpu.VMEM((EPW,), jnp.int32),
        pltpu.VMEM((EPW,), jnp.int32),
        pltpu.VMEM((EPW,), jnp.float32),
    ],
)
def _stats_sc(asrc_hbm, adst_hbm, src_hbm, dst_hbm, exf_hbm, denp_hbm,
              asrc_v, adst_v, den_v, sidx_v, didx_v, ex_v):
    _stats_body(asrc_hbm, adst_hbm, src_hbm, dst_hbm, exf_hbm, denp_hbm,
                asrc_v, adst_v, den_v, sidx_v, didx_v, ex_v)


@functools.partial(
    pl.kernel,
    out_type=jax.ShapeDtypeStruct((NC, N, D), jnp.float32),
    mesh=_sc_mesh,
    compiler_params=_sc_params,
    scratch_types=[
        pltpu.VMEM((CHUNK,), jnp.int32),
        pltpu.VMEM((CHUNK,), jnp.int32),
        pltpu.VMEM((CHUNK,), jnp.int32),
        pltpu.VMEM((CHUNK,), jnp.int32),
        pltpu.VMEM((CHUNK,), jnp.float32),
        pltpu.VMEM((CHUNK,), jnp.float32),
        pltpu.VMEM((CHUNK, D), jnp.float32),
        pltpu.VMEM((CHUNK, D), jnp.float32),
        pltpu.VMEM((CHUNK, D), jnp.float32),
        pltpu.VMEM((CHUNK, D), jnp.float32),
        pltpu.VMEM_SHARED((N, D), jnp.float32),
        pltpu.SemaphoreType.DMA,
        pltpu.SemaphoreType.DMA,
        pltpu.SemaphoreType.DMA,
        pltpu.SemaphoreType.DMA,
        pltpu.SemaphoreType.DMA,
        pltpu.SemaphoreType.DMA,
    ],
)
def _agg_sc(h_hbm, src_hbm, dst_hbm, exf_hbm, zero_hbm, outp_hbm,
            sidx_s0, sidx_s1, didx_s0, didx_s1, ex_s0, ex_s1,
            grow0, grow1, srow0, srow1, shacc,
            gsem0, gsem1, ssem0, ssem1, isem0, isem1):
    _agg_body(h_hbm, src_hbm, dst_hbm, exf_hbm, zero_hbm, outp_hbm,
              sidx_s0, sidx_s1, didx_s0, didx_s1, ex_s0, ex_s1,
              grow0, grow1, srow0, srow1, shacc,
              gsem0, gsem1, ssem0, ssem1, isem0, isem1)


def _matmul_part(z, w_ref, asv_ref, adv_ref, h_ref, asrc_ref, adst_ref):
    h = jnp.dot(z, w_ref[...], preferred_element_type=jnp.float32)
    h_ref[...] = h
    asrc_ref[...] = jnp.dot(h, asv_ref[...], preferred_element_type=jnp.float32)[:, 0]
    adst_ref[...] = jnp.dot(h, adv_ref[...], preferred_element_type=jnp.float32)[:, 0]


def _dense_first_body(z_ref, w_ref, asv_ref, adv_ref, h_ref, asrc_ref, adst_ref):
    _matmul_part(z_ref[...], w_ref, asv_ref, adv_ref, h_ref, asrc_ref, adst_ref)


def _norm_part(outp_ref, denp_ref, b_ref):
    acc = outp_ref[0] + outp_ref[1]
    den = jnp.sum(denp_ref[...], axis=0)
    zp = acc / (den + 1e-16)[:, None] + b_ref[...][None, :]
    mu = jnp.mean(zp, axis=0)
    var = jnp.mean((zp - mu[None, :]) ** 2, axis=0)
    zn = (zp - mu[None, :]) / jnp.sqrt(var + 1e-5)
    return jnp.maximum(zn, 0.0)


def _dense_mid_body(outp_ref, denp_ref, b_ref, w_ref, asv_ref, adv_ref,
                    h_ref, asrc_ref, adst_ref):
    z = _norm_part(outp_ref, denp_ref, b_ref)
    _matmul_part(z, w_ref, asv_ref, adv_ref, h_ref, asrc_ref, adst_ref)


def _dense_last_body(outp_ref, denp_ref, b_ref, z_ref):
    z_ref[...] = _norm_part(outp_ref, denp_ref, b_ref)


_dense_out = (
    jax.ShapeDtypeStruct((N, D), jnp.float32),
    jax.ShapeDtypeStruct((N,), jnp.float32),
    jax.ShapeDtypeStruct((N,), jnp.float32),
)

_dense_first = pl.pallas_call(_dense_first_body, out_shape=_dense_out)
_dense_mid = pl.pallas_call(_dense_mid_body, out_shape=_dense_out)
_dense_last = pl.pallas_call(
    _dense_last_body, out_shape=jax.ShapeDtypeStruct((N, D), jnp.float32))


@jax.jit
def _run(x, src, dst, params):
    zero = jnp.zeros((N, D), jnp.float32)
    (W0, a_s0, a_d0, b0), (W1, a_s1, a_d1, b1), (W2, a_s2, a_d2, b2) = params
    h, asrc, adst = _dense_first(x, W0, a_s0.reshape(D, 1), a_d0.reshape(D, 1))
    exf, denp = _stats_sc(asrc, adst, src, dst)
    outp = _agg_sc(h, src, dst, exf, zero)
    h, asrc, adst = _dense_mid(outp, denp, b0, W1, a_s1.reshape(D, 1),
                               a_d1.reshape(D, 1))
    exf, denp = _stats_sc(asrc, adst, src, dst)
    outp = _agg_sc(h, src, dst, exf, zero)
    h, asrc, adst = _dense_mid(outp, denp, b1, W2, a_s2.reshape(D, 1),
                               a_d2.reshape(D, 1))
    exf, denp = _stats_sc(asrc, adst, src, dst)
    outp = _agg_sc(h, src, dst, exf, zero)
    return _dense_last(outp, denp, b2)


def kernel(x, edge_index, W0, a_src0, a_dst0, b0, W1, a_src1, a_dst1, b1,
           W2, a_src2, a_dst2, b2):
    src = edge_index[0].astype(jnp.int32)
    dst = edge_index[1].astype(jnp.int32)
    params = ((W0, a_src0, a_dst0, b0), (W1, a_src1, a_dst1, b1),
              (W2, a_src2, a_dst2, b2))
    return _run(x, src, dst, params)


# speedup vs baseline: 1.0291x; 1.0291x over previous
import dataclasses
import functools
import jax
import jax.numpy as jnp
from jax import lax
from jax.experimental import pallas as pl
from jax.experimental.pallas import tpu as pltpu
from jax.experimental.pallas import tpu_sc as plsc

N = 10000
E = 320000
D = 128
NC = 2
NS = 16
NW = NC * NS
EPW = E // NW
CHUNK = 80
NCHUNK = EPW // CHUNK
RPT = 624
RREM = N - (NS - 1) * RPT - RPT


_GDN = lax.GatherDimensionNumbers(
    offset_dims=(), collapsed_slice_dims=(0,), start_index_map=(0,))


def _bcast_lane(v, r):
    idx = jnp.full((16, 1), r, jnp.int32)
    return lax.gather(v, idx, _GDN, slice_sizes=(1,),
                      mode=lax.GatherScatterMode.PROMISE_IN_BOUNDS)


def _stats_body(asrc_hbm, adst_hbm, src_hbm, dst_hbm, exf_hbm, denp_hbm,
                asrc_v, adst_v, den_v, sidx_v, didx_v, ex_v):
    cid = lax.axis_index("c")
    sid = lax.axis_index("s")
    wid = cid * NS + sid

    pltpu.sync_copy(asrc_hbm, asrc_v)
    pltpu.sync_copy(adst_hbm, adst_v)
    pltpu.sync_copy(src_hbm.at[pl.ds(wid * EPW, EPW)], sidx_v)
    pltpu.sync_copy(dst_hbm.at[pl.ds(wid * EPW, EPW)], didx_v)

    zeros16 = jnp.zeros((16,), jnp.float32)

    @pl.loop(0, N // 16)
    def _(i):
        den_v[pl.ds(i * 16, 16)] = zeros16

    @pl.loop(0, EPW // 16, unroll=4)
    def _(i):
        si = sidx_v[pl.ds(i * 16, 16)]
        di = didx_v[pl.ds(i * 16, 16)]
        e = plsc.load_gather(asrc_v, [si]) + plsc.load_gather(adst_v, [di])
        e = jnp.maximum(e, 0.2 * e)
        ex = jnp.exp(e)
        plsc.addupdate_scatter(den_v, [di], ex)
        ex_v[pl.ds(i * 16, 16)] = ex

    pltpu.sync_copy(ex_v, exf_hbm.at[pl.ds(wid * EPW, EPW)])
    pltpu.sync_copy(den_v, denp_hbm.at[wid])


def _bcast_lane_dyn(v, r):
    idx = jnp.full((16, 1), r, jnp.int32)
    return lax.gather(v, idx, _GDN, slice_sizes=(1,),
                      mode=lax.GatherScatterMode.PROMISE_IN_BOUNDS)


def _scale_group(exg, grow, srow, g):
    @plsc.parallel_loop(0, 16, unroll=4)
    def _(r):
        k = g * 16 + r
        bc = _bcast_lane_dyn(exg, r)
        for j in range(D // 16):
            srow[k, pl.ds(j * 16, 16)] = grow[k, pl.ds(j * 16, 16)] * bc


def _agg_body(h_hbm, src_hbm, dst_hbm, exf_hbm, outp_hbm,
              sidx_s0, sidx_s1, didx_s0, didx_s1, ex_s0, ex_s1,
              grow0, grow1, srow0, srow1, shacc,
              gsem0, gsem1, ssem0, ssem1, isem0, isem1):
    cid = lax.axis_index("c")
    sid = lax.axis_index("s")
    wid = cid * NS + sid
    sidx_s = (sidx_s0, sidx_s1)
    didx_s = (didx_s0, didx_s1)
    ex_s = (ex_s0, ex_s1)
    grow = (grow0, grow1)
    srow = (srow0, srow1)
    gsem = (gsem0, gsem1)
    ssem = (ssem0, ssem1)
    isem = (isem0, isem1)
    NG = CHUNK // 16
    base0 = wid * EPW

    zeros16 = jnp.zeros((16,), jnp.float32)
    zeros16i = jnp.zeros((16,), jnp.int32)

    @pl.loop(0, CHUNK)
    def _(k):
        for j in range(D // 16):
            srow0[k, pl.ds(j * 16, 16)] = zeros16
            srow1[k, pl.ds(j * 16, 16)] = zeros16

    for q in range(7):
        pltpu.sync_copy(srow0, shacc.at[pl.ds(sid * RPT + q * CHUNK, CHUNK)])
    pltpu.sync_copy(srow0.at[pl.ds(0, 64)],
                    shacc.at[pl.ds(sid * RPT + 7 * CHUNK, 64)])

    @pl.when(sid == NS - 1)
    def _():
        pltpu.sync_copy(srow0.at[pl.ds(0, RREM)],
                        shacc.at[pl.ds(NS * RPT, RREM)])

    for b in range(2):
        pltpu.sync_copy(src_hbm.at[pl.ds(base0 + b * CHUNK, CHUNK)], sidx_s[b])
        pltpu.sync_copy(dst_hbm.at[pl.ds(base0 + b * CHUNK, CHUNK)], didx_s[b])
        pltpu.sync_copy(exf_hbm.at[pl.ds(base0 + b * CHUNK, CHUNK)], ex_s[b])

    plsc.subcore_barrier()

    for b in range(2):
        pltpu.async_copy(h_hbm.at[sidx_s[b]], grow[b], gsem[b])
        for g in range(NG):
            pltpu.async_copy(srow[b].at[pl.ds(g * 16, 16)],
                             shacc.at[zeros16i], ssem[b], add=True)

    def wait_scatter(b):
        for g in range(NG):
            pltpu.make_async_copy(srow[b].at[pl.ds(g * 16, 16)],
                                  shacc.at[zeros16i], ssem[b]).wait()

    def do_chunk(b, t):
        c = 2 * t + b
        ce = base0 + c * CHUNK
        nce = ce + 2 * CHUNK
        more = c + 2 < NCHUNK
        pltpu.make_async_copy(h_hbm.at[sidx_s[b]], grow[b], gsem[b]).wait()

        @pl.when(more)
        def _():
            pltpu.async_copy(src_hbm.at[pl.ds(nce, CHUNK)], sidx_s[b], isem[b])

        wait_scatter(b)
        exg = [ex_s[b][pl.ds(g * 16, 16)] for g in range(NG)]
        dig = [didx_s[b][pl.ds(g * 16, 16)] for g in range(NG)]

        @pl.when(more)
        def _():
            pltpu.async_copy(dst_hbm.at[pl.ds(nce, CHUNK)], didx_s[b], isem[b])
            pltpu.async_copy(exf_hbm.at[pl.ds(nce, CHUNK)], ex_s[b], isem[b])

        for g in range(NG):
            _scale_group(exg[g], grow[b], srow[b], g)
            pltpu.async_copy(srow[b].at[pl.ds(g * 16, 16)],
                             shacc.at[dig[g]], ssem[b], add=True)

        @pl.when(more)
        def _():
            pltpu.make_async_copy(src_hbm.at[pl.ds(nce, CHUNK)], sidx_s[b],
                                  isem[b]).wait()
            pltpu.make_async_copy(dst_hbm.at[pl.ds(nce, CHUNK)], didx_s[b],
                                  isem[b]).wait()
            pltpu.make_async_copy(exf_hbm.at[pl.ds(nce, CHUNK)], ex_s[b],
                                  isem[b]).wait()
            pltpu.async_copy(h_hbm.at[sidx_s[b]], grow[b], gsem[b])

    @pl.loop(0, NCHUNK // 2)
    def _(t):
        do_chunk(0, t)
        do_chunk(1, t)

    do_chunk(0, NCHUNK // 2)

    wait_scatter(1)
    wait_scatter(0)

    plsc.subcore_barrier()
    pltpu.sync_copy(shacc.at[pl.ds(sid * RPT, RPT)],
                    outp_hbm.at[cid].at[pl.ds(sid * RPT, RPT)])

    @pl.when(sid == NS - 1)
    def _():
        pltpu.sync_copy(shacc.at[pl.ds(NS * RPT, RREM)],
                        outp_hbm.at[cid].at[pl.ds(NS * RPT, RREM)])


_sc_params = pltpu.CompilerParams()
if "needs_layout_passes" in pltpu.CompilerParams.__dataclass_fields__:
    _sc_params = dataclasses.replace(_sc_params, needs_layout_passes=False)

_sc_mesh = plsc.VectorSubcoreMesh(core_axis_name="c", subcore_axis_name="s")


@functools.partial(
    pl.kernel,
    out_type=(jax.ShapeDtypeStruct((E,), jnp.float32),
              jax.ShapeDtypeStruct((NW, N), jnp.float32)),
    mesh=_sc_mesh,
    compiler_params=_sc_params,
    scratch_types=[
        pltpu.VMEM((N,), jnp.float32),
        pltpu.VMEM((N,), jnp.float32),
        pltpu.VMEM((N,), jnp.float32),
        pltpu.VMEM((EPW,), jnp.int32),
        pltpu.VMEM((EPW,), jnp.int32),
        pltpu.VMEM((EPW,), jnp.float32),
    ],
)
def _stats_sc(asrc_hbm, adst_hbm, src_hbm, dst_hbm, exf_hbm, denp_hbm,
              asrc_v, adst_v, den_v, sidx_v, didx_v, ex_v):
    _stats_body(asrc_hbm, adst_hbm, src_hbm, dst_hbm, exf_hbm, denp_hbm,
                asrc_v, adst_v, den_v, sidx_v, didx_v, ex_v)


@functools.partial(
    pl.kernel,
    out_type=jax.ShapeDtypeStruct((NC, N, D), jnp.float32),
    mesh=_sc_mesh,
    compiler_params=_sc_params,
    scratch_types=[
        pltpu.VMEM((CHUNK,), jnp.int32),
        pltpu.VMEM((CHUNK,), jnp.int32),
        pltpu.VMEM((CHUNK,), jnp.int32),
        pltpu.VMEM((CHUNK,), jnp.int32),
        pltpu.VMEM((CHUNK,), jnp.float32),
        pltpu.VMEM((CHUNK,), jnp.float32),
        pltpu.VMEM((CHUNK, D), jnp.float32),
        pltpu.VMEM((CHUNK, D), jnp.float32),
        pltpu.VMEM((CHUNK, D), jnp.float32),
        pltpu.VMEM((CHUNK, D), jnp.float32),
        pltpu.VMEM_SHARED((N, D), jnp.float32),
        pltpu.SemaphoreType.DMA,
        pltpu.SemaphoreType.DMA,
        pltpu.SemaphoreType.DMA,
        pltpu.SemaphoreType.DMA,
        pltpu.SemaphoreType.DMA,
        pltpu.SemaphoreType.DMA,
    ],
)
def _agg_sc(h_hbm, src_hbm, dst_hbm, exf_hbm, outp_hbm,
            sidx_s0, sidx_s1, didx_s0, didx_s1, ex_s0, ex_s1,
            grow0, grow1, srow0, srow1, shacc,
            gsem0, gsem1, ssem0, ssem1, isem0, isem1):
    _agg_body(h_hbm, src_hbm, dst_hbm, exf_hbm, outp_hbm,
              sidx_s0, sidx_s1, didx_s0, didx_s1, ex_s0, ex_s1,
              grow0, grow1, srow0, srow1, shacc,
              gsem0, gsem1, ssem0, ssem1, isem0, isem1)


def _matmul_part(z, w_ref, asv_ref, adv_ref, h_ref, asrc_ref, adst_ref):
    h = jnp.dot(z, w_ref[...], preferred_element_type=jnp.float32)
    h_ref[...] = h
    asrc_ref[...] = jnp.dot(h, asv_ref[...], preferred_element_type=jnp.float32)[:, 0]
    adst_ref[...] = jnp.dot(h, adv_ref[...], preferred_element_type=jnp.float32)[:, 0]


def _dense_first_body(z_ref, w_ref, asv_ref, adv_ref, h_ref, asrc_ref, adst_ref):
    _matmul_part(z_ref[...], w_ref, asv_ref, adv_ref, h_ref, asrc_ref, adst_ref)


def _norm_part(outp_ref, denp_ref, b_ref):
    acc = outp_ref[0] + outp_ref[1]
    den = jnp.sum(denp_ref[...], axis=0)
    zp = acc / (den + 1e-16)[:, None] + b_ref[...][None, :]
    mu = jnp.mean(zp, axis=0)
    var = jnp.mean((zp - mu[None, :]) ** 2, axis=0)
    zn = (zp - mu[None, :]) / jnp.sqrt(var + 1e-5)
    return jnp.maximum(zn, 0.0)


def _dense_mid_body(outp_ref, denp_ref, b_ref, w_ref, asv_ref, adv_ref,
                    h_ref, asrc_ref, adst_ref):
    z = _norm_part(outp_ref, denp_ref, b_ref)
    _matmul_part(z, w_ref, asv_ref, adv_ref, h_ref, asrc_ref, adst_ref)


def _dense_last_body(outp_ref, denp_ref, b_ref, z_ref):
    z_ref[...] = _norm_part(outp_ref, denp_ref, b_ref)


_dense_out = (
    jax.ShapeDtypeStruct((N, D), jnp.float32),
    jax.ShapeDtypeStruct((N,), jnp.float32),
    jax.ShapeDtypeStruct((N,), jnp.float32),
)

_dense_first = pl.pallas_call(_dense_first_body, out_shape=_dense_out)
_dense_mid = pl.pallas_call(_dense_mid_body, out_shape=_dense_out)
_dense_last = pl.pallas_call(
    _dense_last_body, out_shape=jax.ShapeDtypeStruct((N, D), jnp.float32))


@jax.jit
def _run(x, src, dst, params):
    (W0, a_s0, a_d0, b0), (W1, a_s1, a_d1, b1), (W2, a_s2, a_d2, b2) = params
    h, asrc, adst = _dense_first(x, W0, a_s0.reshape(D, 1), a_d0.reshape(D, 1))
    exf, denp = _stats_sc(asrc, adst, src, dst)
    outp = _agg_sc(h, src, dst, exf)
    h, asrc, adst = _dense_mid(outp, denp, b0, W1, a_s1.reshape(D, 1),
                               a_d1.reshape(D, 1))
    exf, denp = _stats_sc(asrc, adst, src, dst)
    outp = _agg_sc(h, src, dst, exf)
    h, asrc, adst = _dense_mid(outp, denp, b1, W2, a_s2.reshape(D, 1),
                               a_d2.reshape(D, 1))
    exf, denp = _stats_sc(asrc, adst, src, dst)
    outp = _agg_sc(h, src, dst, exf)
    return _dense_last(outp, denp, b2)


def kernel(x, edge_index, W0, a_src0, a_dst0, b0, W1, a_src1, a_dst1, b1,
           W2, a_src2, a_dst2, b2):
    src = edge_index[0].astype(jnp.int32)
    dst = edge_index[1].astype(jnp.int32)
    params = ((W0, a_src0, a_dst0, b0), (W1, a_src1, a_dst1, b1),
              (W2, a_src2, a_dst2, b2))
    return _run(x, src, dst, params)


# speedup vs baseline: 1.0802x; 1.0497x over previous
import dataclasses
import functools
import jax
import jax.numpy as jnp
from jax import lax
from jax.experimental import pallas as pl
from jax.experimental.pallas import tpu as pltpu
from jax.experimental.pallas import tpu_sc as plsc

N = 10000
E = 320000
D = 128
NC = 2
NS = 16
NW = NC * NS
EPW = E // NW
CHUNK = 80
NCHUNK = EPW // CHUNK
RPT = 624
RREM = N - (NS - 1) * RPT - RPT


_GDN = lax.GatherDimensionNumbers(
    offset_dims=(), collapsed_slice_dims=(0,), start_index_map=(0,))


def _stats_body(asrc_hbm, adst_hbm, src_hbm, dst_hbm, exf_hbm, denp_hbm,
                asrc_v, adst_v, den_v, sidx_v, didx_v, ex_v):
    cid = lax.axis_index("c")
    sid = lax.axis_index("s")
    wid = cid * NS + sid

    pltpu.sync_copy(asrc_hbm, asrc_v)
    pltpu.sync_copy(adst_hbm, adst_v)
    pltpu.sync_copy(src_hbm.at[pl.ds(wid * EPW, EPW)], sidx_v)
    pltpu.sync_copy(dst_hbm.at[pl.ds(wid * EPW, EPW)], didx_v)

    zeros16 = jnp.zeros((16,), jnp.float32)

    @pl.loop(0, N // 16)
    def _(i):
        den_v[pl.ds(i * 16, 16)] = zeros16

    @pl.loop(0, EPW // 16, unroll=8)
    def _(i):
        si = sidx_v[pl.ds(i * 16, 16)]
        di = didx_v[pl.ds(i * 16, 16)]
        e = plsc.load_gather(asrc_v, [si]) + plsc.load_gather(adst_v, [di])
        e = jnp.maximum(e, 0.2 * e)
        ex = jnp.exp(e)
        plsc.addupdate_scatter(den_v, [di], ex)
        ex_v[pl.ds(i * 16, 16)] = ex

    pltpu.sync_copy(ex_v, exf_hbm.at[pl.ds(wid * EPW, EPW)])
    pltpu.sync_copy(den_v, denp_hbm.at[wid])


def _bcast_lane_dyn(v, r):
    idx = jnp.full((16, 1), r, jnp.int32)
    return lax.gather(v, idx, _GDN, slice_sizes=(1,),
                      mode=lax.GatherScatterMode.PROMISE_IN_BOUNDS)


def _scale_group(exg, grow, srow, g):
    @plsc.parallel_loop(0, 16, unroll=8)
    def _(r):
        k = g * 16 + r
        bc = _bcast_lane_dyn(exg, r)
        for j in range(D // 16):
            srow[k, pl.ds(j * 16, 16)] = grow[k, pl.ds(j * 16, 16)] * bc


def _agg_body(h_hbm, src_hbm, dst_hbm, exf_hbm, outp_hbm,
              sidx_s0, sidx_s1, didx_s0, didx_s1, ex_s0, ex_s1,
              grow0, grow1, srow0, srow1, shacc,
              gsem0, gsem1, ssem0, ssem1, isem0, isem1):
    cid = lax.axis_index("c")
    sid = lax.axis_index("s")
    wid = cid * NS + sid
    sidx_s = (sidx_s0, sidx_s1)
    didx_s = (didx_s0, didx_s1)
    ex_s = (ex_s0, ex_s1)
    grow = (grow0, grow1)
    srow = (srow0, srow1)
    gsem = (gsem0, gsem1)
    ssem = (ssem0, ssem1)
    isem = (isem0, isem1)
    NG = CHUNK // 16
    base0 = wid * EPW

    zeros16 = jnp.zeros((16,), jnp.float32)
    zeros16i = jnp.zeros((16,), jnp.int32)

    @pl.loop(0, CHUNK)
    def _(k):
        for j in range(D // 16):
            srow0[k, pl.ds(j * 16, 16)] = zeros16
            srow1[k, pl.ds(j * 16, 16)] = zeros16

    for q in range(7):
        pltpu.sync_copy(srow0, shacc.at[pl.ds(sid * RPT + q * CHUNK, CHUNK)])
    pltpu.sync_copy(srow0.at[pl.ds(0, 64)],
                    shacc.at[pl.ds(sid * RPT + 7 * CHUNK, 64)])

    @pl.when(sid == NS - 1)
    def _():
        pltpu.sync_copy(srow0.at[pl.ds(0, RREM)],
                        shacc.at[pl.ds(NS * RPT, RREM)])

    for b in range(2):
        pltpu.sync_copy(src_hbm.at[pl.ds(base0 + b * CHUNK, CHUNK)], sidx_s[b])
        pltpu.sync_copy(dst_hbm.at[pl.ds(base0 + b * CHUNK, CHUNK)], didx_s[b])
        pltpu.sync_copy(exf_hbm.at[pl.ds(base0 + b * CHUNK, CHUNK)], ex_s[b])

    plsc.subcore_barrier()

    for b in range(2):
        pltpu.async_copy(h_hbm.at[sidx_s[b]], grow[b], gsem[b])
        for g in range(NG):
            pltpu.async_copy(srow[b].at[pl.ds(g * 16, 16)],
                             shacc.at[zeros16i], ssem[b], add=True)

    def wait_scatter(b):
        for g in range(NG):
            pltpu.make_async_copy(srow[b].at[pl.ds(g * 16, 16)],
                                  shacc.at[zeros16i], ssem[b]).wait()

    def do_chunk(b, t):
        c = 2 * t + b
        ce = base0 + c * CHUNK
        nce = ce + 2 * CHUNK
        more = c + 2 < NCHUNK
        pltpu.make_async_copy(h_hbm.at[sidx_s[b]], grow[b], gsem[b]).wait()

        @pl.when(more)
        def _():
            pltpu.async_copy(src_hbm.at[pl.ds(nce, CHUNK)], sidx_s[b], isem[b])

        wait_scatter(b)
        exg = [ex_s[b][pl.ds(g * 16, 16)] for g in range(NG)]
        dig = [didx_s[b][pl.ds(g * 16, 16)] for g in range(NG)]

        @pl.when(more)
        def _():
            pltpu.async_copy(dst_hbm.at[pl.ds(nce, CHUNK)], didx_s[b], isem[b])
            pltpu.async_copy(exf_hbm.at[pl.ds(nce, CHUNK)], ex_s[b], isem[b])

        for g in range(NG):
            _scale_group(exg[g], grow[b], srow[b], g)
            pltpu.async_copy(srow[b].at[pl.ds(g * 16, 16)],
                             shacc.at[dig[g]], ssem[b], add=True)

        @pl.when(more)
        def _():
            pltpu.make_async_copy(src_hbm.at[pl.ds(nce, CHUNK)], sidx_s[b],
                                  isem[b]).wait()
            pltpu.make_async_copy(dst_hbm.at[pl.ds(nce, CHUNK)], didx_s[b],
                                  isem[b]).wait()
            pltpu.make_async_copy(exf_hbm.at[pl.ds(nce, CHUNK)], ex_s[b],
                                  isem[b]).wait()
            pltpu.async_copy(h_hbm.at[sidx_s[b]], grow[b], gsem[b])

    @pl.loop(0, NCHUNK // 2)
    def _(t):
        do_chunk(0, t)
        do_chunk(1, t)

    do_chunk(0, NCHUNK // 2)

    wait_scatter(1)
    wait_scatter(0)

    plsc.subcore_barrier()
    pltpu.sync_copy(shacc.at[pl.ds(sid * RPT, RPT)],
                    outp_hbm.at[cid].at[pl.ds(sid * RPT, RPT)])

    @pl.when(sid == NS - 1)
    def _():
        pltpu.sync_copy(shacc.at[pl.ds(NS * RPT, RREM)],
                        outp_hbm.at[cid].at[pl.ds(NS * RPT, RREM)])


_sc_params = pltpu.CompilerParams()
if "needs_layout_passes" in pltpu.CompilerParams.__dataclass_fields__:
    _sc_params = dataclasses.replace(_sc_params, needs_layout_passes=False)

_sc_mesh = plsc.VectorSubcoreMesh(core_axis_name="c", subcore_axis_name="s")


@functools.partial(
    pl.kernel,
    out_type=(jax.ShapeDtypeStruct((E,), jnp.float32),
              jax.ShapeDtypeStruct((NW, N), jnp.float32)),
    mesh=_sc_mesh,
    compiler_params=_sc_params,
    scratch_types=[
        pltpu.VMEM((N,), jnp.float32),
        pltpu.VMEM((N,), jnp.float32),
        pltpu.VMEM((N,), jnp.float32),
        pltpu.VMEM((EPW,), jnp.int32),
        pltpu.VMEM((EPW,), jnp.int32),
        pltpu.VMEM((EPW,), jnp.float32),
    ],
)
def _stats_sc(asrc_hbm, adst_hbm, src_hbm, dst_hbm, exf_hbm, denp_hbm,
              asrc_v, adst_v, den_v, sidx_v, didx_v, ex_v):
    _stats_body(asrc_hbm, adst_hbm, src_hbm, dst_hbm, exf_hbm, denp_hbm,
                asrc_v, adst_v, den_v, sidx_v, didx_v, ex_v)


@functools.partial(
    pl.kernel,
    out_type=jax.ShapeDtypeStruct((NC, N, D), jnp.float32),
    mesh=_sc_mesh,
    compiler_params=_sc_params,
    scratch_types=[
        pltpu.VMEM((CHUNK,), jnp.int32),
        pltpu.VMEM((CHUNK,), jnp.int32),
        pltpu.VMEM((CHUNK,), jnp.int32),
        pltpu.VMEM((CHUNK,), jnp.int32),
        pltpu.VMEM((CHUNK,), jnp.float32),
        pltpu.VMEM((CHUNK,), jnp.float32),
        pltpu.VMEM((CHUNK, D), jnp.float32),
        pltpu.VMEM((CHUNK, D), jnp.float32),
        pltpu.VMEM((CHUNK, D), jnp.float32),
        pltpu.VMEM((CHUNK, D), jnp.float32),
        pltpu.VMEM_SHARED((N, D), jnp.float32),
        pltpu.SemaphoreType.DMA,
        pltpu.SemaphoreType.DMA,
        pltpu.SemaphoreType.DMA,
        pltpu.SemaphoreType.DMA,
        pltpu.SemaphoreType.DMA,
        pltpu.SemaphoreType.DMA,
    ],
)
def _agg_sc(h_hbm, src_hbm, dst_hbm, exf_hbm, outp_hbm,
            sidx_s0, sidx_s1, didx_s0, didx_s1, ex_s0, ex_s1,
            grow0, grow1, srow0, srow1, shacc,
            gsem0, gsem1, ssem0, ssem1, isem0, isem1):
    _agg_body(h_hbm, src_hbm, dst_hbm, exf_hbm, outp_hbm,
              sidx_s0, sidx_s1, didx_s0, didx_s1, ex_s0, ex_s1,
              grow0, grow1, srow0, srow1, shacc,
              gsem0, gsem1, ssem0, ssem1, isem0, isem1)


def _matmul_part(z, w_ref, asv_ref, adv_ref, h_ref, asrc_ref, adst_ref):
    h = jnp.dot(z, w_ref[...], preferred_element_type=jnp.float32)
    h_ref[...] = h
    asrc_ref[...] = jnp.dot(h, asv_ref[...], preferred_element_type=jnp.float32)[:, 0]
    adst_ref[...] = jnp.dot(h, adv_ref[...], preferred_element_type=jnp.float32)[:, 0]


def _dense_first_body(z_ref, w_ref, asv_ref, adv_ref, h_ref, asrc_ref, adst_ref):
    _matmul_part(z_ref[...], w_ref, asv_ref, adv_ref, h_ref, asrc_ref, adst_ref)


def _norm_part(outp_ref, denp_ref, b_ref):
    acc = outp_ref[0] + outp_ref[1]
    den = jnp.sum(denp_ref[...], axis=0)
    zp = acc / (den + 1e-16)[:, None] + b_ref[...][None, :]
    mu = jnp.mean(zp, axis=0)
    var = jnp.mean((zp - mu[None, :]) ** 2, axis=0)
    zn = (zp - mu[None, :]) / jnp.sqrt(var + 1e-5)
    return jnp.maximum(zn, 0.0)


def _dense_mid_body(outp_ref, denp_ref, b_ref, w_ref, asv_ref, adv_ref,
                    h_ref, asrc_ref, adst_ref):
    z = _norm_part(outp_ref, denp_ref, b_ref)
    _matmul_part(z, w_ref, asv_ref, adv_ref, h_ref, asrc_ref, adst_ref)


def _dense_last_body(outp_ref, denp_ref, b_ref, z_ref):
    z_ref[...] = _norm_part(outp_ref, denp_ref, b_ref)


_dense_out = (
    jax.ShapeDtypeStruct((N, D), jnp.float32),
    jax.ShapeDtypeStruct((N,), jnp.float32),
    jax.ShapeDtypeStruct((N,), jnp.float32),
)

_dense_first = pl.pallas_call(_dense_first_body, out_shape=_dense_out)
_dense_mid = pl.pallas_call(_dense_mid_body, out_shape=_dense_out)
_dense_last = pl.pallas_call(
    _dense_last_body, out_shape=jax.ShapeDtypeStruct((N, D), jnp.float32))


@jax.jit
def _run(x, src, dst, params):
    (W0, a_s0, a_d0, b0), (W1, a_s1, a_d1, b1), (W2, a_s2, a_d2, b2) = params
    h, asrc, adst = _dense_first(x, W0, a_s0.reshape(D, 1), a_d0.reshape(D, 1))
    exf, denp = _stats_sc(asrc, adst, src, dst)
    outp = _agg_sc(h, src, dst, exf)
    h, asrc, adst = _dense_mid(outp, denp, b0, W1, a_s1.reshape(D, 1),
                               a_d1.reshape(D, 1))
    exf, denp = _stats_sc(asrc, adst, src, dst)
    outp = _agg_sc(h, src, dst, exf)
    h, asrc, adst = _dense_mid(outp, denp, b1, W2, a_s2.reshape(D, 1),
                               a_d2.reshape(D, 1))
    exf, denp = _stats_sc(asrc, adst, src, dst)
    outp = _agg_sc(h, src, dst, exf)
    return _dense_last(outp, denp, b2)


def kernel(x, edge_index, W0, a_src0, a_dst0, b0, W1, a_src1, a_dst1, b1,
           W2, a_src2, a_dst2, b2):
    src = edge_index[0].astype(jnp.int32)
    dst = edge_index[1].astype(jnp.int32)
    params = ((W0, a_src0, a_dst0, b0), (W1, a_src1, a_dst1, b1),
              (W2, a_src2, a_dst2, b2))
    return _run(x, src, dst, params)
